# baseline (speedup 1.0000x reference)
import jax
import jax.numpy as jnp
from jax import lax
from jax.experimental import pallas as pl
from jax.experimental.pallas import tpu as pltpu
from jax.experimental.pallas import tpu_sc as plsc

NA = 64
ROW_F = 2 * NA
B = 16384
L = 16
NW = 32
PER_W = B // NW
CHUNK = 256
N_CHUNKS = PER_W // CHUNK


_GATHER_DNUMS = lax.GatherDimensionNumbers(
    offset_dims=(), collapsed_slice_dims=(0,), start_index_map=(0,))


def _permute16(x, idx):
  return lax.gather(
      x, idx[:, None], _GATHER_DNUMS, slice_sizes=(1,),
      mode=lax.GatherScatterMode.PROMISE_IN_BOUNDS)


def _rsqrt16(x):
  xi = lax.bitcast_convert_type(x, jnp.int32)
  yi = jnp.int32(0x5F3759DF) - (xi >> 1)
  y = lax.bitcast_convert_type(yi, jnp.float32)
  xh = x * jnp.float32(0.5)
  for _ in range(3):
    y = y * (jnp.float32(1.5) - xh * y * y)
  return y


def _sc_body(idx_hbm, cr_hbm, ci_hbm, out_hbm,
             idx_v, rows_r, rows_i, outb, sem_r, sem_i):
  nc = 2
  wid = lax.axis_index("s") * nc + lax.axis_index("c")
  base = wid * PER_W
  iota = lax.iota(jnp.int32, L)

  def chunk_body(c, carry):
    cbase = base + c * CHUNK
    pltpu.sync_copy(idx_hbm.at[pl.ds(cbase, CHUNK)], idx_v)
    cp_r = pltpu.async_copy(cr_hbm.at[idx_v], rows_r, sem_r)
    cp_i = pltpu.async_copy(ci_hbm.at[idx_v], rows_i, sem_i)
    cp_r.wait()
    cp_i.wait()

    def row_body(j, carry2):
      rs = [rows_r[j, pl.ds(L * k, L)] for k in range(NA // L)]
      im = [rows_i[j, pl.ds(L * k, L)] for k in range(NA // L)]
      ss = rs[0] * rs[0]
      for v in rs[1:]:
        ss = ss + v * v
      for v in im:
        ss = ss + v * v
      for sh in (8, 4, 2, 1):
        ss = ss + _permute16(ss, iota ^ sh)
      scale = _rsqrt16(ss)
      rowbase = j * ROW_F
      for k in range(NA // L):
        col = rowbase + (iota + (L * k)) * 2
        plsc.store_scatter(outb, [col], rs[k] * scale)
        plsc.store_scatter(outb, [col + 1], im[k] * scale)
      return carry2

    lax.fori_loop(0, CHUNK, row_body, 0, unroll=2)
    pltpu.sync_copy(outb, out_hbm.at[pl.ds(cbase * ROW_F, CHUNK * ROW_F)])
    return carry

  lax.fori_loop(0, N_CHUNKS, chunk_body, 0)


@jax.jit
def _sc_call(beam_index, codebook_real, codebook_imag):
  mesh = plsc.VectorSubcoreMesh(core_axis_name="c", subcore_axis_name="s")
  f = pl.kernel(
      _sc_body,
      out_type=jax.ShapeDtypeStruct((B * ROW_F,), jnp.float32),
      mesh=mesh,
      compiler_params=pltpu.CompilerParams(
          needs_layout_passes=False, use_tc_tiling_on_sc=False),
      scratch_types=[
          pltpu.VMEM((CHUNK,), jnp.int32),
          pltpu.VMEM((CHUNK, NA), jnp.float32),
          pltpu.VMEM((CHUNK, NA), jnp.float32),
          pltpu.VMEM((CHUNK * ROW_F,), jnp.float32),
          pltpu.SemaphoreType.DMA,
          pltpu.SemaphoreType.DMA,
      ],
  )
  return f(beam_index, codebook_real, codebook_imag)


def kernel(beam_index, codebook_real, codebook_imag):
  out = _sc_call(beam_index, codebook_real, codebook_imag)
  return out.reshape(B, NA, 2)


# speedup vs baseline: 1.0458x; 1.0458x over previous
import jax
import jax.numpy as jnp
from jax import lax
from jax.experimental import pallas as pl
from jax.experimental.pallas import tpu as pltpu
from jax.experimental.pallas import tpu_sc as plsc

NA = 64
ROW_F = 2 * NA
B = 16384
L = 16
NW = 32
PER_W = B // NW
CHUNK = 256
N_CHUNKS = PER_W // CHUNK


_GATHER_DNUMS = lax.GatherDimensionNumbers(
    offset_dims=(), collapsed_slice_dims=(0,), start_index_map=(0,))


def _permute16(x, idx):
  return lax.gather(
      x, idx[:, None], _GATHER_DNUMS, slice_sizes=(1,),
      mode=lax.GatherScatterMode.PROMISE_IN_BOUNDS)


def _rsqrt16(x):
  xi = lax.bitcast_convert_type(x, jnp.int32)
  yi = jnp.int32(0x5F3759DF) - (xi >> 1)
  y = lax.bitcast_convert_type(yi, jnp.float32)
  xh = x * jnp.float32(0.5)
  for _ in range(3):
    y = y * (jnp.float32(1.5) - xh * y * y)
  return y


def _sc_body(idx_hbm, cr_hbm, ci_hbm, out_hbm,
             idx_v, rows_r, rows_i, outb, sem_r, sem_i):
  nc = 2
  wid = lax.axis_index("s") * nc + lax.axis_index("c")
  base = wid * PER_W
  iota = lax.iota(jnp.int32, L)

  def chunk_body(c, carry):
    cbase = base + c * CHUNK
    pltpu.sync_copy(idx_hbm.at[pl.ds(cbase, CHUNK)], idx_v)

    def fire(t, carry2):
      gvec = idx_v[pl.ds(t * L, L)]
      for lane in range(L):
        g = gvec[lane]
        j = t * L + lane
        pltpu.make_async_copy(cr_hbm.at[pl.ds(g, 1), :],
                              rows_r.at[pl.ds(j, 1), :], sem_r).start()
        pltpu.make_async_copy(ci_hbm.at[pl.ds(g, 1), :],
                              rows_i.at[pl.ds(j, 1), :], sem_i).start()
      return carry2

    lax.fori_loop(0, CHUNK // L, fire, 0)
    pltpu.make_async_copy(cr_hbm.at[pl.ds(0, CHUNK), :], rows_r,
                          sem_r).wait()
    pltpu.make_async_copy(ci_hbm.at[pl.ds(0, CHUNK), :], rows_i,
                          sem_i).wait()

    def row_body(j, carry2):
      rs = [rows_r[j, pl.ds(L * k, L)] for k in range(NA // L)]
      im = [rows_i[j, pl.ds(L * k, L)] for k in range(NA // L)]
      ss = rs[0] * rs[0]
      for v in rs[1:]:
        ss = ss + v * v
      for v in im:
        ss = ss + v * v
      for sh in (8, 4, 2, 1):
        ss = ss + _permute16(ss, iota ^ sh)
      scale = _rsqrt16(ss)
      rowbase = j * ROW_F
      for k in range(NA // L):
        col = rowbase + (iota + (L * k)) * 2
        plsc.store_scatter(outb, [col], rs[k] * scale)
        plsc.store_scatter(outb, [col + 1], im[k] * scale)
      return carry2

    lax.fori_loop(0, CHUNK, row_body, 0, unroll=2)
    pltpu.sync_copy(outb, out_hbm.at[pl.ds(cbase * ROW_F, CHUNK * ROW_F)])
    return carry

  lax.fori_loop(0, N_CHUNKS, chunk_body, 0)


@jax.jit
def _sc_call(beam_index, codebook_real, codebook_imag):
  mesh = plsc.VectorSubcoreMesh(core_axis_name="c", subcore_axis_name="s")
  f = pl.kernel(
      _sc_body,
      out_type=jax.ShapeDtypeStruct((B * ROW_F,), jnp.float32),
      mesh=mesh,
      compiler_params=pltpu.CompilerParams(
          needs_layout_passes=False, use_tc_tiling_on_sc=True),
      scratch_types=[
          pltpu.VMEM((CHUNK,), jnp.int32),
          pltpu.VMEM((CHUNK, NA), jnp.float32),
          pltpu.VMEM((CHUNK, NA), jnp.float32),
          pltpu.VMEM((CHUNK * ROW_F,), jnp.float32),
          pltpu.SemaphoreType.DMA,
          pltpu.SemaphoreType.DMA,
      ],
  )
  return f(beam_index, codebook_real, codebook_imag)


def kernel(beam_index, codebook_real, codebook_imag):
  out = _sc_call(beam_index, codebook_real, codebook_imag)
  return out.reshape(B, NA, 2)


# speedup vs baseline: 5.5542x; 5.3112x over previous
import jax
import jax.numpy as jnp
from jax import lax
from jax.experimental import pallas as pl
from jax.experimental.pallas import tpu as pltpu
from jax.experimental.pallas import tpu_sc as plsc

NA = 64
ROW_F = 2 * NA
B = 16384
L = 16
NW = 32
PER_W = B // NW
CHUNK = 256
N_CHUNKS = PER_W // CHUNK
BLK = 128
NBLK = B // BLK


_GATHER_DNUMS = lax.GatherDimensionNumbers(
    offset_dims=(), collapsed_slice_dims=(0,), start_index_map=(0,))


def _permute16(x, idx):
  return lax.gather(
      x, idx[:, None], _GATHER_DNUMS, slice_sizes=(1,),
      mode=lax.GatherScatterMode.PROMISE_IN_BOUNDS)


def _rsqrt16(x):
  xi = lax.bitcast_convert_type(x, jnp.int32)
  yi = jnp.int32(0x5F3759DF) - (xi >> 1)
  y = lax.bitcast_convert_type(yi, jnp.float32)
  xh = x * jnp.float32(0.5)
  for _ in range(3):
    y = y * (jnp.float32(1.5) - xh * y * y)
  return y


def _sc_body(idx_hbm, cr_hbm, ci_hbm, out_hbm,
             idx_v, rows_r, rows_i, outb, sem_r, sem_i, sem_o):
  nc = 2
  wid = lax.axis_index("s") * nc + lax.axis_index("c")
  base = wid * PER_W
  iota = lax.iota(jnp.int32, L)

  def chunk_body(c, carry):
    cbase = base + c * CHUNK
    pltpu.sync_copy(idx_hbm.at[pl.ds(cbase, CHUNK)], idx_v)

    def fire(t, carry2):
      gvec = idx_v[pl.ds(t * L, L)]
      for lane in range(L):
        g = gvec[lane]
        j = t * L + lane
        pltpu.make_async_copy(cr_hbm.at[pl.ds(g, 1), :],
                              rows_r.at[pl.ds(j, 1), :], sem_r).start()
        pltpu.make_async_copy(ci_hbm.at[pl.ds(g, 1), :],
                              rows_i.at[pl.ds(j, 1), :], sem_i).start()
      return carry2

    lax.fori_loop(0, CHUNK // L, fire, 0)
    pltpu.make_async_copy(cr_hbm.at[pl.ds(0, CHUNK), :], rows_r,
                          sem_r).wait()
    pltpu.make_async_copy(ci_hbm.at[pl.ds(0, CHUNK), :], rows_i,
                          sem_i).wait()

    def row_body(j, carry2):
      rs = [rows_r[j, pl.ds(L * k, L)] for k in range(NA // L)]
      im = [rows_i[j, pl.ds(L * k, L)] for k in range(NA // L)]
      ss = rs[0] * rs[0]
      for v in rs[1:]:
        ss = ss + v * v
      for v in im:
        ss = ss + v * v
      for sh in (8, 4, 2, 1):
        ss = ss + _permute16(ss, iota ^ sh)
      scale = _rsqrt16(ss)
      tl = j >> 7
      s = j & 127
      colbase = tl * (ROW_F * BLK) + s
      for k in range(NA // L):
        col = colbase + (iota + (L * k)) * (2 * BLK)
        plsc.store_scatter(outb, [col], rs[k] * scale)
        plsc.store_scatter(outb, [col + BLK], im[k] * scale)
      return carry2

    lax.fori_loop(0, CHUNK, row_body, 0, unroll=2)
    for tl in range(CHUNK // BLK):
      tglob = wid * (PER_W // BLK) + c * (CHUNK // BLK) + tl
      for a in range(NA):
        pltpu.make_async_copy(
            outb.at[pl.ds(tl * (ROW_F * BLK) + a * (2 * BLK), 2 * BLK)],
            out_hbm.at[pl.ds(a * (2 * BLK * NBLK) + tglob * (2 * BLK),
                             2 * BLK)],
            sem_o).start()
    pltpu.make_async_copy(out_hbm.at[pl.ds(0, CHUNK * ROW_F)], outb,
                          sem_o).wait()
    return carry

  lax.fori_loop(0, N_CHUNKS, chunk_body, 0)


@jax.jit
def _sc_call(beam_index, codebook_real, codebook_imag):
  mesh = plsc.VectorSubcoreMesh(core_axis_name="c", subcore_axis_name="s")
  f = pl.kernel(
      _sc_body,
      out_type=jax.ShapeDtypeStruct((B * ROW_F,), jnp.float32),
      mesh=mesh,
      compiler_params=pltpu.CompilerParams(
          needs_layout_passes=False, use_tc_tiling_on_sc=True),
      scratch_types=[
          pltpu.VMEM((CHUNK,), jnp.int32),
          pltpu.VMEM((CHUNK, NA), jnp.float32),
          pltpu.VMEM((CHUNK, NA), jnp.float32),
          pltpu.VMEM((CHUNK * ROW_F,), jnp.float32),
          pltpu.SemaphoreType.DMA,
          pltpu.SemaphoreType.DMA,
          pltpu.SemaphoreType.DMA,
      ],
  )
  return f(beam_index, codebook_real, codebook_imag)


def kernel(beam_index, codebook_real, codebook_imag):
  out = _sc_call(beam_index, codebook_real, codebook_imag)
  out = out.reshape(NA, NBLK, 2, BLK)
  out = out.transpose(1, 3, 0, 2)
  return out.reshape(B, NA, 2)


# speedup vs baseline: 5.6749x; 1.0217x over previous
import jax
import jax.numpy as jnp
from jax import lax
from jax.experimental import pallas as pl
from jax.experimental.pallas import tpu as pltpu
from jax.experimental.pallas import tpu_sc as plsc

NA = 64
ROW_F = 2 * NA
B = 16384
L = 16
NW = 32
PER_W = B // NW
BLK = 128
NBLK = B // BLK
CHUNK = BLK
N_CHUNKS = PER_W // CHUNK


_GATHER_DNUMS = lax.GatherDimensionNumbers(
    offset_dims=(), collapsed_slice_dims=(0,), start_index_map=(0,))


def _permute16(x, idx):
  return lax.gather(
      x, idx[:, None], _GATHER_DNUMS, slice_sizes=(1,),
      mode=lax.GatherScatterMode.PROMISE_IN_BOUNDS)


def _rsqrt16(x):
  xi = lax.bitcast_convert_type(x, jnp.int32)
  yi = jnp.int32(0x5F3759DF) - (xi >> 1)
  y = lax.bitcast_convert_type(yi, jnp.float32)
  xh = x * jnp.float32(0.5)
  for _ in range(3):
    y = y * (jnp.float32(1.5) - xh * y * y)
  return y


def _sc_body(idx_hbm, cr_hbm, ci_hbm, out_hbm,
             idx_v, rows_r0, rows_i0, rows_r1, rows_i1, outb0, outb1,
             sem_r0, sem_i0, sem_r1, sem_i1, sem_o0, sem_o1):
  nc = 2
  wid = lax.axis_index("s") * nc + lax.axis_index("c")
  base = wid * PER_W
  iota = lax.iota(jnp.int32, L)

  rows_r = (rows_r0, rows_r1)
  rows_i = (rows_i0, rows_i1)
  outb = (outb0, outb1)
  sem_r = (sem_r0, sem_r1)
  sem_i = (sem_i0, sem_i1)
  sem_o = (sem_o0, sem_o1)

  pltpu.sync_copy(idx_hbm.at[pl.ds(base, PER_W)], idx_v)

  def fire_gather(c):
    p = c % 2

    def fire(t, carry):
      gvec = idx_v[pl.ds(c * CHUNK + t * L, L)]
      for lane in range(L):
        g = gvec[lane]
        j = t * L + lane
        pltpu.make_async_copy(cr_hbm.at[pl.ds(g, 1), :],
                              rows_r[p].at[pl.ds(j, 1), :], sem_r[p]).start()
        pltpu.make_async_copy(ci_hbm.at[pl.ds(g, 1), :],
                              rows_i[p].at[pl.ds(j, 1), :], sem_i[p]).start()
      return carry

    lax.fori_loop(0, CHUNK // L, fire, 0)

  def drain_gather(c):
    p = c % 2
    pltpu.make_async_copy(cr_hbm.at[pl.ds(0, CHUNK), :], rows_r[p],
                          sem_r[p]).wait()
    pltpu.make_async_copy(ci_hbm.at[pl.ds(0, CHUNK), :], rows_i[p],
                          sem_i[p]).wait()

  def compute(c):
    p = c % 2

    def row_body(j, carry):
      rs = [rows_r[p][j, pl.ds(L * k, L)] for k in range(NA // L)]
      im = [rows_i[p][j, pl.ds(L * k, L)] for k in range(NA // L)]
      ss = rs[0] * rs[0]
      for v in rs[1:]:
        ss = ss + v * v
      for v in im:
        ss = ss + v * v
      for sh in (8, 4, 2, 1):
        ss = ss + _permute16(ss, iota ^ sh)
      scale = _rsqrt16(ss)
      for k in range(NA // L):
        col = j + (iota + (L * k)) * (2 * BLK)
        plsc.store_scatter(outb[p], [col], rs[k] * scale)
        plsc.store_scatter(outb[p], [col + BLK], im[k] * scale)
      return carry

    lax.fori_loop(0, CHUNK, row_body, 0, unroll=2)

  def fire_out(c):
    p = c % 2
    tglob = wid * N_CHUNKS + c
    for a in range(NA):
      pltpu.make_async_copy(
          outb[p].at[pl.ds(a * (2 * BLK), 2 * BLK)],
          out_hbm.at[pl.ds(a * (2 * BLK * NBLK) + tglob * (2 * BLK),
                           2 * BLK)],
          sem_o[p]).start()

  def drain_out(c):
    p = c % 2
    pltpu.make_async_copy(out_hbm.at[pl.ds(0, ROW_F * BLK)], outb[p],
                          sem_o[p]).wait()

  fire_gather(0)
  for c in range(N_CHUNKS):
    if c + 1 < N_CHUNKS:
      fire_gather(c + 1)
    drain_gather(c)
    if c >= 2:
      drain_out(c - 2)
    compute(c)
    fire_out(c)
  drain_out(N_CHUNKS - 2)
  drain_out(N_CHUNKS - 1)


@jax.jit
def _sc_call(beam_index, codebook_real, codebook_imag):
  mesh = plsc.VectorSubcoreMesh(core_axis_name="c", subcore_axis_name="s")
  f = pl.kernel(
      _sc_body,
      out_type=jax.ShapeDtypeStruct((B * ROW_F,), jnp.float32),
      mesh=mesh,
      compiler_params=pltpu.CompilerParams(
          needs_layout_passes=False, use_tc_tiling_on_sc=True),
      scratch_types=[
          pltpu.VMEM((PER_W,), jnp.int32),
          pltpu.VMEM((CHUNK, NA), jnp.float32),
          pltpu.VMEM((CHUNK, NA), jnp.float32),
          pltpu.VMEM((CHUNK, NA), jnp.float32),
          pltpu.VMEM((CHUNK, NA), jnp.float32),
          pltpu.VMEM((ROW_F * BLK,), jnp.float32),
          pltpu.VMEM((ROW_F * BLK,), jnp.float32),
          pltpu.SemaphoreType.DMA,
          pltpu.SemaphoreType.DMA,
          pltpu.SemaphoreType.DMA,
          pltpu.SemaphoreType.DMA,
          pltpu.SemaphoreType.DMA,
          pltpu.SemaphoreType.DMA,
      ],
  )
  return f(beam_index, codebook_real, codebook_imag)


def kernel(beam_index, codebook_real, codebook_imag):
  out = _sc_call(beam_index, codebook_real, codebook_imag)
  out = out.reshape(NA, NBLK, 2, BLK)
  out = out.transpose(1, 3, 0, 2)
  return out.reshape(B, NA, 2)
